# baseline (speedup 1.0000x reference)
import functools

import jax
import jax.numpy as jnp
from jax import lax
from jax.experimental import pallas as pl
from jax.experimental.pallas import tpu as pltpu
from jax.experimental.pallas import tpu_sc as plsc

LAMBDA_PARAM = 25.0
MU_PARAM = 25.0
NU_PARAM = 1.0
ALPHA = 0.75
EPS = 1e-4
K_FEAT = 20
K_GRID = 4
SLOTS = 64
C_FEAT = 96
C_PAD = 128


def _row_sq_norms(x):
    ones = jnp.ones((1, x.shape[1]), dtype=x.dtype)
    return lax.dot_general(ones, x * x, (((1,), (1,)), ((), ())),
                           preferred_element_type=jnp.float32)


def _min_argmin_ax0(m, n, ridx_f):
    mn = jnp.min(m, axis=0, keepdims=True)
    arg = jnp.min(jnp.where(m == mn, ridx_f, jnp.float32(n)), axis=0,
                  keepdims=True)
    return mn, arg


BPP = 4


def _select_body(za_ref, zb_ref, ga_ref, gb_ref,
                 idxa_ref, idxb_ref, tpa_ref, tpb_ref):
    n = za_ref.shape[1]
    pid = pl.program_id(0)

    iota_s = lax.broadcasted_iota(jnp.int32, (1, SLOTS), 1)
    cidx = (lax.broadcasted_iota(jnp.int32, (1, n), 1) * n
            ).astype(jnp.float32)
    ridx_f = lax.broadcasted_iota(jnp.int32, (n, n), 0).astype(jnp.float32)

    def sq_oriented(rows, cols):
        r2 = jnp.sum(rows * rows, axis=1, keepdims=True)
        c2 = _row_sq_norms(cols)
        dot = lax.dot_general(rows, cols, (((1,), (1,)), ((), ())),
                              preferred_element_type=jnp.float32)
        return r2 + c2 - 2.0 * dot

    def min_pack(rows, cols):
        mn, arg = _min_argmin_ax0(sq_oriented(rows, cols), n, ridx_f)
        return [mn, cidx + arg]

    chains = []
    accs = []
    lane_pad = ((0, 0), (0, C_PAD - C_FEAT))
    for lb in range(BPP):
        za = za_ref[lb]
        zb = zb_ref[lb]
        ga = ga_ref[lb]
        gb = gb_ref[lb]
        tpa_ref[pl.ds(lb * n, n), :] = jnp.pad(za, lane_pad)
        tpb_ref[pl.ds(lb * n, n), :] = jnp.pad(zb, lane_pad)
        chains.append(min_pack(zb, za) + [K_FEAT, 0, False, lb])
        chains.append(min_pack(za, zb) + [K_FEAT, K_FEAT, True, lb])
        chains.append(min_pack(gb, ga) + [K_GRID, 2 * K_FEAT, False, lb])
        chains.append(min_pack(ga, gb)
                      + [K_GRID, 2 * K_FEAT + K_GRID, True, lb])
        accs.append([jnp.zeros((1, SLOTS), jnp.int32),
                     jnp.zeros((1, SLOTS), jnp.int32)])

    big = jnp.float32(n * n)
    inf = jnp.float32(jnp.inf)
    for t in range(K_FEAT):
        live = [st for st in chains if t < st[2]]
        ms = [jnp.min(st[0]) for st in live]
        combs = [jnp.min(jnp.where(st[0] == m, st[1], big))
                 for st, m in zip(live, ms)]
        for st, comb in zip(live, combs):
            vals, pack, k, slot0, swapped, lb = st
            combi = comb.astype(jnp.int32)
            sel = combi // n
            nn = combi - sel * n
            hit = iota_s == (slot0 + t)
            acc_a, acc_b = accs[lb]
            if swapped:
                acc_b = jnp.where(hit, sel, acc_b)
                acc_a = jnp.where(hit, nn, acc_a)
            else:
                acc_a = jnp.where(hit, sel, acc_a)
                acc_b = jnp.where(hit, nn, acc_b)
            accs[lb] = [acc_a, acc_b]
            st[0] = jnp.where(pack == comb, inf, vals)

    for lb in range(BPP):
        off = (pid * BPP + lb) * n
        idxa_ref[lb] = accs[lb][0] + off
        idxb_ref[lb] = accs[lb][1] + off


def _select_call(za, zb, ga, gb):
    bsz, n, c = za.shape
    gc = ga.shape[-1]
    return pl.pallas_call(
        _select_body,
        grid=(bsz // BPP,),
        in_specs=[
            pl.BlockSpec((BPP, n, c), lambda b: (b, 0, 0)),
            pl.BlockSpec((BPP, n, c), lambda b: (b, 0, 0)),
            pl.BlockSpec((BPP, n, gc), lambda b: (b, 0, 0)),
            pl.BlockSpec((BPP, n, gc), lambda b: (b, 0, 0)),
        ],
        out_specs=[
            pl.BlockSpec((BPP, 1, SLOTS), lambda b: (b, 0, 0)),
            pl.BlockSpec((BPP, 1, SLOTS), lambda b: (b, 0, 0)),
            pl.BlockSpec((BPP * n, C_PAD), lambda b: (b, 0)),
            pl.BlockSpec((BPP * n, C_PAD), lambda b: (b, 0)),
        ],
        out_shape=[
            jax.ShapeDtypeStruct((bsz, 1, SLOTS), jnp.int32),
            jax.ShapeDtypeStruct((bsz, 1, SLOTS), jnp.int32),
            jax.ShapeDtypeStruct((bsz * n, C_PAD), jnp.float32),
            jax.ShapeDtypeStruct((bsz * n, C_PAD), jnp.float32),
        ],
    )(za, zb, ga, gb)


def _gather_call(tab_a, tab_b, idx_a, idx_b):
    tot, c = idx_a.shape[0], tab_a.shape[1]
    info = plsc.get_sparse_core_info()
    nc, ns = info.num_cores, info.num_subcores
    rpw = tot // (nc * ns)
    mesh = plsc.VectorSubcoreMesh(core_axis_name="c", subcore_axis_name="s")

    @functools.partial(
        pl.kernel, mesh=mesh,
        out_type=[jax.ShapeDtypeStruct((tot, c), jnp.float32),
                  jax.ShapeDtypeStruct((tot, c), jnp.float32)],
        scratch_types=[pltpu.VMEM((rpw,), jnp.int32),
                       pltpu.VMEM((rpw, c), jnp.float32),
                       pltpu.SemaphoreType.DMA],
    )
    def k(ta, tb, ia, ib, oa, ob, idx_v, rows_v, sem):
        wid = lax.axis_index("s") * nc + lax.axis_index("c")
        base = wid * rpw
        pltpu.sync_copy(ia.at[pl.ds(base, rpw)], idx_v)
        pltpu.async_copy(ta.at[idx_v], rows_v, sem).wait()
        pltpu.sync_copy(rows_v, oa.at[pl.ds(base, rpw)])
        pltpu.sync_copy(ib.at[pl.ds(base, rpw)], idx_v)
        pltpu.async_copy(tb.at[idx_v], rows_v, sem).wait()
        pltpu.sync_copy(rows_v, ob.at[pl.ds(base, rpw)])

    return k(tab_a, tab_b, idx_a, idx_b)


def _vicreg_terms(xs, ys):
    bsz = xs[0].shape[0]
    c = xs[0].shape[1]
    m_cnt = len(xs)
    bm1 = jnp.float32(bsz - 1)
    inv_sum = jnp.float32(0.0)
    var_sum = jnp.float32(0.0)
    cov_sum = jnp.float32(0.0)
    for xm, ym in zip(xs, ys):
        d = xm - ym
        inv_sum = inv_sum + jnp.sum(d * d)
        for zm in (xm, ym):
            mu = jnp.sum(zm, axis=0, keepdims=True) / bsz
            zc = zm - mu
            dvec = jnp.sum(zc * zc, axis=0, keepdims=True)
            std = jnp.sqrt(dvec / bm1 + EPS)
            var_sum = var_sum + jnp.sum(jnp.maximum(1.0 - std, 0.0))
            gram = lax.dot_general(zc, zc, (((1,), (1,)), ((), ())),
                                   preferred_element_type=jnp.float32)
            cov_sum = cov_sum + (jnp.sum(gram * gram)
                                 - jnp.sum(dvec * dvec))
    inv = inv_sum / (m_cnt * bsz * c)
    var = 0.5 * var_sum / (m_cnt * c)
    cov = 0.5 * cov_sum / (m_cnt * c * bm1 * bm1)
    return LAMBDA_PARAM * inv + MU_PARAM * var + NU_PARAM * cov


def _loss_body(outa_ref, outb_ref, zag_ref, zbg_ref, out_ref):
    def sl(ref, lo, k):
        return [ref[:, lo + t, 0:C_FEAT] for t in range(k)]

    fa = sl(outa_ref, 0, K_FEAT)
    fbn = sl(outa_ref, K_FEAT, K_FEAT)
    fal = sl(outa_ref, 2 * K_FEAT, K_GRID)
    fbnl = sl(outa_ref, 2 * K_FEAT + K_GRID, K_GRID)
    fan = sl(outb_ref, 0, K_FEAT)
    fb = sl(outb_ref, K_FEAT, K_FEAT)
    fanl = sl(outb_ref, 2 * K_FEAT, K_GRID)
    fbl = sl(outb_ref, 2 * K_FEAT + K_GRID, K_GRID)

    l2 = 0.5 * (_vicreg_terms(fa, fan) + _vicreg_terms(fb, fbn))
    loc = 0.5 * (_vicreg_terms(fal, fanl) + _vicreg_terms(fbl, fbnl))
    glob = _vicreg_terms([zag_ref[...]], [zbg_ref[...]])
    total = ALPHA * glob + (1.0 - ALPHA) * (l2 + loc)
    out_ref[...] = jnp.broadcast_to(total, (1, 1))


def _loss_call(outa, outb, zag, zbg):
    return pl.pallas_call(
        _loss_body,
        out_shape=jax.ShapeDtypeStruct((1, 1), jnp.float32),
    )(outa, outb, zag, zbg)


def kernel(z_a_global, z_b_global, z_a_local, z_b_local, grid_a, grid_b):
    bsz = z_a_local.shape[0]
    c = z_a_local.shape[-1]
    gc = grid_a.shape[-1]
    n = z_a_local.shape[1] * z_a_local.shape[2]
    za = z_a_local.reshape(bsz, n, c)
    zb = z_b_local.reshape(bsz, n, c)
    ga = grid_a.reshape(bsz, n, gc)
    gb = grid_b.reshape(bsz, n, gc)
    idxa, idxb, tapa, tapb = _select_call(za, zb, ga, gb)
    outa, outb = _gather_call(tapa, tapb,
                              idxa.reshape(bsz * SLOTS),
                              idxb.reshape(bsz * SLOTS))
    total = _loss_call(outa.reshape(bsz, SLOTS, C_PAD),
                       outb.reshape(bsz, SLOTS, C_PAD),
                       z_a_global, z_b_global)
    return total[0, 0]


# speedup vs baseline: 1.0123x; 1.0123x over previous
import functools

import jax
import jax.numpy as jnp
from jax import lax
from jax.experimental import pallas as pl
from jax.experimental.pallas import tpu as pltpu
from jax.experimental.pallas import tpu_sc as plsc

LAMBDA_PARAM = 25.0
MU_PARAM = 25.0
NU_PARAM = 1.0
ALPHA = 0.75
EPS = 1e-4
K_FEAT = 20
K_GRID = 4
SLOTS = 64
C_FEAT = 96
C_PAD = 128


def _row_sq_norms(x):
    ones = jnp.ones((1, x.shape[1]), dtype=x.dtype)
    return lax.dot_general(ones, x * x, (((1,), (1,)), ((), ())),
                           preferred_element_type=jnp.float32)


def _min_argmin_ax0(m, n, ridx_f):
    mn = jnp.min(m, axis=0, keepdims=True)
    arg = jnp.min(jnp.where(m == mn, ridx_f, jnp.float32(n)), axis=0,
                  keepdims=True)
    return mn, arg


BPP = 2


def _select_body(za_ref, zb_ref, ga_ref, gb_ref,
                 idxa_ref, idxb_ref, tpa_ref, tpb_ref):
    n = za_ref.shape[1]
    pid = pl.program_id(0)

    iota_s = lax.broadcasted_iota(jnp.int32, (1, SLOTS), 1)
    cidx = (lax.broadcasted_iota(jnp.int32, (1, n), 1) * n
            ).astype(jnp.float32)
    ridx_f = lax.broadcasted_iota(jnp.int32, (n, n), 0).astype(jnp.float32)

    def min_pack(rows, cols):
        r2col = jnp.sum(rows * rows, axis=1, keepdims=True)
        c2row = _row_sq_norms(cols)
        mp = lax.dot_general(rows * -2.0, cols, (((1,), (1,)), ((), ())),
                             preferred_element_type=jnp.float32) + r2col
        mn, arg = _min_argmin_ax0(mp, n, ridx_f)
        return [mn + c2row, cidx + arg]

    def min_pack_t(rowsT, colsT):
        gc = rowsT.shape[0]
        ones_c1 = jnp.ones((gc, 1), jnp.float32)
        ones_1c = jnp.ones((1, gc), jnp.float32)
        r2col = lax.dot_general(rowsT * rowsT, ones_c1,
                                (((0,), (0,)), ((), ())),
                                preferred_element_type=jnp.float32)
        c2row = lax.dot_general(ones_1c, colsT * colsT,
                                (((1,), (0,)), ((), ())),
                                preferred_element_type=jnp.float32)
        mp = lax.dot_general(rowsT * -2.0, colsT, (((0,), (0,)), ((), ())),
                             preferred_element_type=jnp.float32) + r2col
        mn, arg = _min_argmin_ax0(mp, n, ridx_f)
        return [mn + c2row, cidx + arg]

    chains = []
    accs = []
    lane_pad = ((0, 0), (0, C_PAD - C_FEAT))
    for lb in range(BPP):
        za = za_ref[lb]
        zb = zb_ref[lb]
        ga = ga_ref[lb]
        gb = gb_ref[lb]
        tpa_ref[pl.ds(lb * n, n), :] = jnp.pad(za, lane_pad)
        tpb_ref[pl.ds(lb * n, n), :] = jnp.pad(zb, lane_pad)
        chains.append(min_pack(zb, za) + [K_FEAT, 0, False, lb])
        chains.append(min_pack(za, zb) + [K_FEAT, K_FEAT, True, lb])
        chains.append(min_pack_t(gb, ga) + [K_GRID, 2 * K_FEAT, False, lb])
        chains.append(min_pack_t(ga, gb)
                      + [K_GRID, 2 * K_FEAT + K_GRID, True, lb])
        accs.append([jnp.zeros((1, SLOTS), jnp.int32),
                     jnp.zeros((1, SLOTS), jnp.int32)])

    big = jnp.float32(n * n)
    inf = jnp.float32(jnp.inf)
    for t in range(K_FEAT):
        live = [st for st in chains if t < st[2]]
        ms = [jnp.min(st[0]) for st in live]
        combs = [jnp.min(jnp.where(st[0] == m, st[1], big))
                 for st, m in zip(live, ms)]
        for st, comb in zip(live, combs):
            vals, pack, k, slot0, swapped, lb = st
            combi = comb.astype(jnp.int32)
            sel = combi // n
            nn = combi - sel * n
            hit = iota_s == (slot0 + t)
            acc_a, acc_b = accs[lb]
            if swapped:
                acc_b = jnp.where(hit, sel, acc_b)
                acc_a = jnp.where(hit, nn, acc_a)
            else:
                acc_a = jnp.where(hit, sel, acc_a)
                acc_b = jnp.where(hit, nn, acc_b)
            accs[lb] = [acc_a, acc_b]
            st[0] = jnp.where(pack == comb, inf, vals)

    for lb in range(BPP):
        off = (pid * BPP + lb) * n
        idxa_ref[lb] = accs[lb][0] + off
        idxb_ref[lb] = accs[lb][1] + off


def _select_call(za, zb, ga, gb):
    bsz, n, c = za.shape
    gc = ga.shape[1]
    return pl.pallas_call(
        _select_body,
        grid=(bsz // BPP,),
        in_specs=[
            pl.BlockSpec((BPP, n, c), lambda b: (b, 0, 0)),
            pl.BlockSpec((BPP, n, c), lambda b: (b, 0, 0)),
            pl.BlockSpec((BPP, gc, n), lambda b: (b, 0, 0)),
            pl.BlockSpec((BPP, gc, n), lambda b: (b, 0, 0)),
        ],
        out_specs=[
            pl.BlockSpec((BPP, 1, SLOTS), lambda b: (b, 0, 0)),
            pl.BlockSpec((BPP, 1, SLOTS), lambda b: (b, 0, 0)),
            pl.BlockSpec((BPP * n, C_PAD), lambda b: (b, 0)),
            pl.BlockSpec((BPP * n, C_PAD), lambda b: (b, 0)),
        ],
        out_shape=[
            jax.ShapeDtypeStruct((bsz, 1, SLOTS), jnp.int32),
            jax.ShapeDtypeStruct((bsz, 1, SLOTS), jnp.int32),
            jax.ShapeDtypeStruct((bsz * n, C_PAD), jnp.float32),
            jax.ShapeDtypeStruct((bsz * n, C_PAD), jnp.float32),
        ],
    )(za, zb, ga, gb)


def _gather_call(tab_a, tab_b, idx_a, idx_b):
    tot, c = idx_a.shape[0], tab_a.shape[1]
    info = plsc.get_sparse_core_info()
    nc, ns = info.num_cores, info.num_subcores
    rpw = tot // (nc * ns)
    mesh = plsc.VectorSubcoreMesh(core_axis_name="c", subcore_axis_name="s")

    @functools.partial(
        pl.kernel, mesh=mesh,
        out_type=[jax.ShapeDtypeStruct((tot, c), jnp.float32),
                  jax.ShapeDtypeStruct((tot, c), jnp.float32)],
        scratch_types=[pltpu.VMEM((rpw,), jnp.int32),
                       pltpu.VMEM((rpw, c), jnp.float32),
                       pltpu.SemaphoreType.DMA],
    )
    def k(ta, tb, ia, ib, oa, ob, idx_v, rows_v, sem):
        wid = lax.axis_index("s") * nc + lax.axis_index("c")
        base = wid * rpw
        pltpu.sync_copy(ia.at[pl.ds(base, rpw)], idx_v)
        pltpu.async_copy(ta.at[idx_v], rows_v, sem).wait()
        pltpu.sync_copy(rows_v, oa.at[pl.ds(base, rpw)])
        pltpu.sync_copy(ib.at[pl.ds(base, rpw)], idx_v)
        pltpu.async_copy(tb.at[idx_v], rows_v, sem).wait()
        pltpu.sync_copy(rows_v, ob.at[pl.ds(base, rpw)])

    return k(tab_a, tab_b, idx_a, idx_b)


def _vicreg_terms(xs, ys):
    bsz = xs[0].shape[0]
    c = xs[0].shape[1]
    m_cnt = len(xs)
    bm1 = jnp.float32(bsz - 1)
    inv_sum = jnp.float32(0.0)
    var_sum = jnp.float32(0.0)
    cov_sum = jnp.float32(0.0)
    for xm, ym in zip(xs, ys):
        d = xm - ym
        inv_sum = inv_sum + jnp.sum(d * d)
        for zm in (xm, ym):
            mu = jnp.sum(zm, axis=0, keepdims=True) / bsz
            zc = zm - mu
            dvec = jnp.sum(zc * zc, axis=0, keepdims=True)
            std = jnp.sqrt(dvec / bm1 + EPS)
            var_sum = var_sum + jnp.sum(jnp.maximum(1.0 - std, 0.0))
            gram = lax.dot_general(zc, zc, (((1,), (1,)), ((), ())),
                                   preferred_element_type=jnp.float32)
            cov_sum = cov_sum + (jnp.sum(gram * gram)
                                 - jnp.sum(dvec * dvec))
    inv = inv_sum / (m_cnt * bsz * c)
    var = 0.5 * var_sum / (m_cnt * c)
    cov = 0.5 * cov_sum / (m_cnt * c * bm1 * bm1)
    return LAMBDA_PARAM * inv + MU_PARAM * var + NU_PARAM * cov


def _loss_body(outa_ref, outb_ref, zag_ref, zbg_ref, out_ref):
    def sl(ref, lo, k):
        return [ref[:, lo + t, 0:C_FEAT] for t in range(k)]

    fa = sl(outa_ref, 0, K_FEAT)
    fbn = sl(outa_ref, K_FEAT, K_FEAT)
    fal = sl(outa_ref, 2 * K_FEAT, K_GRID)
    fbnl = sl(outa_ref, 2 * K_FEAT + K_GRID, K_GRID)
    fan = sl(outb_ref, 0, K_FEAT)
    fb = sl(outb_ref, K_FEAT, K_FEAT)
    fanl = sl(outb_ref, 2 * K_FEAT, K_GRID)
    fbl = sl(outb_ref, 2 * K_FEAT + K_GRID, K_GRID)

    l2 = 0.5 * (_vicreg_terms(fa, fan) + _vicreg_terms(fb, fbn))
    loc = 0.5 * (_vicreg_terms(fal, fanl) + _vicreg_terms(fbl, fbnl))
    glob = _vicreg_terms([zag_ref[...]], [zbg_ref[...]])
    total = ALPHA * glob + (1.0 - ALPHA) * (l2 + loc)
    out_ref[...] = jnp.broadcast_to(total, (1, 1))


def _loss_call(outa, outb, zag, zbg):
    return pl.pallas_call(
        _loss_body,
        out_shape=jax.ShapeDtypeStruct((1, 1), jnp.float32),
    )(outa, outb, zag, zbg)


def kernel(z_a_global, z_b_global, z_a_local, z_b_local, grid_a, grid_b):
    bsz = z_a_local.shape[0]
    c = z_a_local.shape[-1]
    gc = grid_a.shape[-1]
    n = z_a_local.shape[1] * z_a_local.shape[2]
    za = z_a_local.reshape(bsz, n, c)
    zb = z_b_local.reshape(bsz, n, c)
    ga = jnp.swapaxes(grid_a.reshape(bsz, n, gc), 1, 2)
    gb = jnp.swapaxes(grid_b.reshape(bsz, n, gc), 1, 2)
    idxa, idxb, tapa, tapb = _select_call(za, zb, ga, gb)
    outa, outb = _gather_call(tapa, tapb,
                              idxa.reshape(bsz * SLOTS),
                              idxb.reshape(bsz * SLOTS))
    total = _loss_call(outa.reshape(bsz, SLOTS, C_PAD),
                       outb.reshape(bsz, SLOTS, C_PAD),
                       z_a_global, z_b_global)
    return total[0, 0]


# speedup vs baseline: 1.0234x; 1.0109x over previous
import functools

import jax
import jax.numpy as jnp
from jax import lax
from jax.experimental import pallas as pl
from jax.experimental.pallas import tpu as pltpu
from jax.experimental.pallas import tpu_sc as plsc

LAMBDA_PARAM = 25.0
MU_PARAM = 25.0
NU_PARAM = 1.0
ALPHA = 0.75
EPS = 1e-4
K_FEAT = 20
K_GRID = 4
SLOTS = 64
C_FEAT = 96
C_PAD = 128


def _row_sq_norms(x):
    ones = jnp.ones((1, x.shape[1]), dtype=x.dtype)
    return lax.dot_general(ones, x * x, (((1,), (1,)), ((), ())),
                           preferred_element_type=jnp.float32)


def _min_argmin_ax0(m, n, ridx_f):
    mn = jnp.min(m, axis=0, keepdims=True)
    arg = jnp.min(jnp.where(m == mn, ridx_f, jnp.float32(n)), axis=0,
                  keepdims=True)
    return mn, arg


BPP = 2


def _select_body(za_ref, zb_ref, ga_ref, gb_ref,
                 idxa_ref, idxb_ref, tpa_ref, tpb_ref):
    n = za_ref.shape[1]
    pid = pl.program_id(0)

    iota_s = lax.broadcasted_iota(jnp.int32, (1, SLOTS), 1)
    cidx = (lax.broadcasted_iota(jnp.int32, (1, n), 1) * n
            ).astype(jnp.float32)
    ridx_f = lax.broadcasted_iota(jnp.int32, (n, n), 0).astype(jnp.float32)

    def min_pack(rows, cols):
        r2col = jnp.sum(rows * rows, axis=1, keepdims=True)
        c2row = _row_sq_norms(cols)
        mp = lax.dot_general(rows * -2.0, cols, (((1,), (1,)), ((), ())),
                             preferred_element_type=jnp.float32) + r2col
        mn, arg = _min_argmin_ax0(mp, n, ridx_f)
        return [mn + c2row, cidx + arg]

    def min_pack_t(rowsT, colsT):
        gc = rowsT.shape[0]
        ones_c1 = jnp.ones((gc, 1), jnp.float32)
        ones_1c = jnp.ones((1, gc), jnp.float32)
        r2col = lax.dot_general(rowsT * rowsT, ones_c1,
                                (((0,), (0,)), ((), ())),
                                preferred_element_type=jnp.float32)
        c2row = lax.dot_general(ones_1c, colsT * colsT,
                                (((1,), (0,)), ((), ())),
                                preferred_element_type=jnp.float32)
        mp = lax.dot_general(rowsT * -2.0, colsT, (((0,), (0,)), ((), ())),
                             preferred_element_type=jnp.float32) + r2col
        mn, arg = _min_argmin_ax0(mp, n, ridx_f)
        return [mn + c2row, cidx + arg]

    chains = []
    accs = []
    lane_pad = ((0, 0), (0, C_PAD - C_FEAT))
    for lb in range(BPP):
        za = za_ref[lb]
        zb = zb_ref[lb]
        ga = ga_ref[lb]
        gb = gb_ref[lb]
        tpa_ref[pl.ds(lb * n, n), :] = jnp.pad(za, lane_pad)
        tpb_ref[pl.ds(lb * n, n), :] = jnp.pad(zb, lane_pad)
        chains.append(min_pack(zb, za) + [K_FEAT, 0, False, lb])
        chains.append(min_pack(za, zb) + [K_FEAT, K_FEAT, True, lb])
        chains.append(min_pack_t(gb, ga) + [K_GRID, 2 * K_FEAT, False, lb])
        chains.append(min_pack_t(ga, gb)
                      + [K_GRID, 2 * K_FEAT + K_GRID, True, lb])
        accs.append([jnp.zeros((1, SLOTS), jnp.int32),
                     jnp.zeros((1, SLOTS), jnp.int32)])

    big = jnp.float32(n * n)
    inf = jnp.float32(jnp.inf)
    for t in range(K_FEAT):
        live = [st for st in chains if t < st[2]]
        ms = [jnp.min(st[0]) for st in live]
        combs = [jnp.min(jnp.where(st[0] == m, st[1], big))
                 for st, m in zip(live, ms)]
        for st, comb in zip(live, combs):
            vals, pack, k, slot0, swapped, lb = st
            combi = comb.astype(jnp.int32)
            sel = combi // n
            nn = combi - sel * n
            hit = iota_s == (slot0 + t)
            acc_a, acc_b = accs[lb]
            if swapped:
                acc_b = jnp.where(hit, sel, acc_b)
                acc_a = jnp.where(hit, nn, acc_a)
            else:
                acc_a = jnp.where(hit, sel, acc_a)
                acc_b = jnp.where(hit, nn, acc_b)
            accs[lb] = [acc_a, acc_b]
            st[0] = jnp.where(pack == comb, inf, vals)

    for lb in range(BPP):
        off = (pid * BPP + lb) * n
        idxa_ref[lb] = accs[lb][0] + off
        idxb_ref[lb] = accs[lb][1] + off


def _select_call(za, zb, ga, gb):
    bsz, n, c = za.shape
    gc = ga.shape[1]
    return pl.pallas_call(
        _select_body,
        grid=(bsz // BPP,),
        in_specs=[
            pl.BlockSpec((BPP, n, c), lambda b: (b, 0, 0)),
            pl.BlockSpec((BPP, n, c), lambda b: (b, 0, 0)),
            pl.BlockSpec((BPP, gc, n), lambda b: (b, 0, 0)),
            pl.BlockSpec((BPP, gc, n), lambda b: (b, 0, 0)),
        ],
        out_specs=[
            pl.BlockSpec((BPP, 1, SLOTS), lambda b: (b, 0, 0)),
            pl.BlockSpec((BPP, 1, SLOTS), lambda b: (b, 0, 0)),
            pl.BlockSpec((BPP * n, C_PAD), lambda b: (b, 0)),
            pl.BlockSpec((BPP * n, C_PAD), lambda b: (b, 0)),
        ],
        out_shape=[
            jax.ShapeDtypeStruct((bsz, 1, SLOTS), jnp.int32),
            jax.ShapeDtypeStruct((bsz, 1, SLOTS), jnp.int32),
            jax.ShapeDtypeStruct((bsz * n, C_PAD), jnp.float32),
            jax.ShapeDtypeStruct((bsz * n, C_PAD), jnp.float32),
        ],
    )(za, zb, ga, gb)


def _gather_call(tab_a, tab_b, idx_a, idx_b):
    tot, c = idx_a.shape[0], tab_a.shape[1]
    info = plsc.get_sparse_core_info()
    nc, ns = info.num_cores, info.num_subcores
    rpw = tot // (nc * ns)
    mesh = plsc.VectorSubcoreMesh(core_axis_name="c", subcore_axis_name="s")

    @functools.partial(
        pl.kernel, mesh=mesh,
        out_type=[jax.ShapeDtypeStruct((tot, c), jnp.float32),
                  jax.ShapeDtypeStruct((tot, c), jnp.float32)],
        scratch_types=[pltpu.VMEM((rpw,), jnp.int32),
                       pltpu.VMEM((rpw,), jnp.int32),
                       pltpu.VMEM((rpw, c), jnp.float32),
                       pltpu.VMEM((rpw, c), jnp.float32),
                       pltpu.SemaphoreType.DMA,
                       pltpu.SemaphoreType.DMA],
    )
    def k(ta, tb, ia, ib, oa, ob, ixa_v, ixb_v, rwa_v, rwb_v, sema, semb):
        wid = lax.axis_index("s") * nc + lax.axis_index("c")
        base = wid * rpw
        c0 = pltpu.async_copy(ia.at[pl.ds(base, rpw)], ixa_v, sema)
        c1 = pltpu.async_copy(ib.at[pl.ds(base, rpw)], ixb_v, semb)
        c0.wait()
        c1.wait()
        g0 = pltpu.async_copy(ta.at[ixa_v], rwa_v, sema)
        g1 = pltpu.async_copy(tb.at[ixb_v], rwb_v, semb)
        g0.wait()
        g1.wait()
        w0 = pltpu.async_copy(rwa_v, oa.at[pl.ds(base, rpw)], sema)
        w1 = pltpu.async_copy(rwb_v, ob.at[pl.ds(base, rpw)], semb)
        w0.wait()
        w1.wait()

    return k(tab_a, tab_b, idx_a, idx_b)


def _vicreg_terms(xs, ys):
    bsz = xs[0].shape[0]
    c = xs[0].shape[1]
    m_cnt = len(xs)
    bm1 = jnp.float32(bsz - 1)
    inv_sum = jnp.float32(0.0)
    var_sum = jnp.float32(0.0)
    cov_sum = jnp.float32(0.0)
    for xm, ym in zip(xs, ys):
        d = xm - ym
        inv_sum = inv_sum + jnp.sum(d * d)
        for zm in (xm, ym):
            mu = jnp.sum(zm, axis=0, keepdims=True) / bsz
            zc = zm - mu
            dvec = jnp.sum(zc * zc, axis=0, keepdims=True)
            std = jnp.sqrt(dvec / bm1 + EPS)
            var_sum = var_sum + jnp.sum(jnp.maximum(1.0 - std, 0.0))
            gram = lax.dot_general(zc, zc, (((1,), (1,)), ((), ())),
                                   preferred_element_type=jnp.float32)
            cov_sum = cov_sum + (jnp.sum(gram * gram)
                                 - jnp.sum(dvec * dvec))
    inv = inv_sum / (m_cnt * bsz * c)
    var = 0.5 * var_sum / (m_cnt * c)
    cov = 0.5 * cov_sum / (m_cnt * c * bm1 * bm1)
    return LAMBDA_PARAM * inv + MU_PARAM * var + NU_PARAM * cov


def _loss_body(outa_ref, outb_ref, zag_ref, zbg_ref, out_ref):
    def sl(ref, lo, k):
        return [ref[:, lo + t, 0:C_FEAT] for t in range(k)]

    fa = sl(outa_ref, 0, K_FEAT)
    fbn = sl(outa_ref, K_FEAT, K_FEAT)
    fal = sl(outa_ref, 2 * K_FEAT, K_GRID)
    fbnl = sl(outa_ref, 2 * K_FEAT + K_GRID, K_GRID)
    fan = sl(outb_ref, 0, K_FEAT)
    fb = sl(outb_ref, K_FEAT, K_FEAT)
    fanl = sl(outb_ref, 2 * K_FEAT, K_GRID)
    fbl = sl(outb_ref, 2 * K_FEAT + K_GRID, K_GRID)

    l2 = 0.5 * (_vicreg_terms(fa, fan) + _vicreg_terms(fb, fbn))
    loc = 0.5 * (_vicreg_terms(fal, fanl) + _vicreg_terms(fbl, fbnl))
    glob = _vicreg_terms([zag_ref[...]], [zbg_ref[...]])
    total = ALPHA * glob + (1.0 - ALPHA) * (l2 + loc)
    out_ref[...] = jnp.broadcast_to(total, (1, 1))


def _loss_call(outa, outb, zag, zbg):
    return pl.pallas_call(
        _loss_body,
        out_shape=jax.ShapeDtypeStruct((1, 1), jnp.float32),
    )(outa, outb, zag, zbg)


def kernel(z_a_global, z_b_global, z_a_local, z_b_local, grid_a, grid_b):
    bsz = z_a_local.shape[0]
    c = z_a_local.shape[-1]
    gc = grid_a.shape[-1]
    n = z_a_local.shape[1] * z_a_local.shape[2]
    za = z_a_local.reshape(bsz, n, c)
    zb = z_b_local.reshape(bsz, n, c)
    ga = jnp.swapaxes(grid_a.reshape(bsz, n, gc), 1, 2)
    gb = jnp.swapaxes(grid_b.reshape(bsz, n, gc), 1, 2)
    idxa, idxb, tapa, tapb = _select_call(za, zb, ga, gb)
    outa, outb = _gather_call(tapa, tapb,
                              idxa.reshape(bsz * SLOTS),
                              idxb.reshape(bsz * SLOTS))
    total = _loss_call(outa.reshape(bsz, SLOTS, C_PAD),
                       outb.reshape(bsz, SLOTS, C_PAD),
                       z_a_global, z_b_global)
    return total[0, 0]
